# baseline (speedup 1.0000x reference)
import functools

import jax
import jax.numpy as jnp
from jax.experimental import pallas as pl
from jax.experimental.pallas import tpu as pltpu

_N_C, _N_L, _D = 10000, 4096, 128
_CHUNK = 1000
_N_CHUNKS = _N_C // _CHUNK
_N_BUF = 2
_QL = _N_L // 4


def _fused_body(adj_ref, xc_ref, hid_ref, wih_ref, whh_ref, bih_ref,
                bhh_ref, h_ref, c_ref, bufs_ref, xcb_ref, hid_buf,
                acc_ref, gpart_ref, hbuf_ref, cbuf_ref, sems_ref,
                xsems_ref, hsem_ref, csem_ref, osems_ref):
    def start(i):
        slot = i % _N_BUF if isinstance(i, int) else jax.lax.rem(i, _N_BUF)
        pltpu.make_async_copy(
            adj_ref.at[pl.ds(i * _CHUNK, _CHUNK), :],
            bufs_ref.at[slot], sems_ref.at[slot]).start()
        pltpu.make_async_copy(
            xc_ref.at[pl.ds(i * _CHUNK, _CHUNK), :],
            xcb_ref.at[slot], xsems_ref.at[slot]).start()

    xl_cp = pltpu.make_async_copy(hid_ref.at[0], hid_buf.at[0], hsem_ref)
    xl_cp.start()
    for i in range(_N_BUF):
        start(i)
    xl_cp.wait()

    def mm_t(a, w):
        return jax.lax.dot_general(
            a, w, dimension_numbers=(((1,), (1,)), ((), ())),
            preferred_element_type=jnp.float32)

    bias = bih_ref[...] + bhh_ref[...]
    w_flip = wih_ref[:, _D:]

    for q in range(4):
        rows = pl.ds(q * _QL, _QL)
        frows = pl.ds(((q + 2) % 4) * _QL, _QL)
        gpart_ref[rows, :] = mm_t(hid_buf[0, frows, :], w_flip) \
            + mm_t(hid_buf[0, rows, :], whh_ref[...]) + bias
    pltpu.make_async_copy(
        adj_ref.at[pl.ds(0, _CHUNK), :], bufs_ref.at[0],
        sems_ref.at[0]).wait()
    pltpu.make_async_copy(
        xc_ref.at[pl.ds(0, _CHUNK), :], xcb_ref.at[0],
        xsems_ref.at[0]).wait()
    acc_ref[...] = jax.lax.dot_general(
        bufs_ref[0], xcb_ref[0],
        dimension_numbers=(((0,), (0,)), ((), ())),
        preferred_element_type=jnp.float32)
    start(_N_BUF)

    def step(i, _):
        slot = jax.lax.rem(i, _N_BUF)
        pltpu.make_async_copy(
            adj_ref.at[pl.ds(i * _CHUNK, _CHUNK), :],
            bufs_ref.at[slot], sems_ref.at[slot]).wait()
        pltpu.make_async_copy(
            xc_ref.at[pl.ds(i * _CHUNK, _CHUNK), :],
            xcb_ref.at[slot], xsems_ref.at[slot]).wait()
        acc_ref[...] += jax.lax.dot_general(
            bufs_ref[slot], xcb_ref[slot],
            dimension_numbers=(((0,), (0,)), ((), ())),
            preferred_element_type=jnp.float32)

        @pl.when(i + _N_BUF < _N_CHUNKS)
        def _refill():
            start(i + _N_BUF)

        @pl.when(i == _N_CHUNKS - _N_BUF - 1)
        def _c0():
            pltpu.make_async_copy(
                hid_ref.at[1], hid_buf.at[1], csem_ref).start()
        return _

    jax.lax.fori_loop(1, _N_CHUNKS - 1, step, 0)

    last_slot = (_N_CHUNKS - 1) % _N_BUF
    pltpu.make_async_copy(
        adj_ref.at[pl.ds((_N_CHUNKS - 1) * _CHUNK, _CHUNK), :],
        bufs_ref.at[last_slot], sems_ref.at[last_slot]).wait()
    pltpu.make_async_copy(
        xc_ref.at[pl.ds((_N_CHUNKS - 1) * _CHUNK, _CHUNK), :],
        xcb_ref.at[last_slot], xsems_ref.at[last_slot]).wait()

    w_msg = wih_ref[:, :_D]
    for q in range(4):
        rows = pl.ds(q * _QL, _QL)
        msg_q = acc_ref[rows, :] + jax.lax.dot_general(
            bufs_ref[last_slot, :, rows], xcb_ref[last_slot],
            dimension_numbers=(((0,), (0,)), ((), ())),
            preferred_element_type=jnp.float32)
        gates = gpart_ref[rows, :] + mm_t(msg_q, w_msg)
        i_g = jax.nn.sigmoid(gates[:, :_D])
        f_g = jax.nn.sigmoid(gates[:, _D:2 * _D])
        g_g = jnp.tanh(gates[:, 2 * _D:3 * _D])
        o_g = jax.nn.sigmoid(gates[:, 3 * _D:])
        if q == 0:
            pltpu.make_async_copy(
                hid_ref.at[1], hid_buf.at[1], csem_ref).wait()
        c = f_g * hid_buf[1, rows, :] + i_g * g_g
        slot = q % 2
        if q >= 2:
            prows = pl.ds((q - 2) * _QL, _QL)
            pltpu.make_async_copy(hbuf_ref.at[slot], h_ref.at[prows, :],
                                  osems_ref.at[2 * slot]).wait()
            pltpu.make_async_copy(cbuf_ref.at[slot], c_ref.at[prows, :],
                                  osems_ref.at[2 * slot + 1]).wait()
        hbuf_ref[slot] = o_g * jnp.tanh(c)
        cbuf_ref[slot] = c
        pltpu.make_async_copy(hbuf_ref.at[slot], h_ref.at[rows, :],
                              osems_ref.at[2 * slot]).start()
        pltpu.make_async_copy(cbuf_ref.at[slot], c_ref.at[rows, :],
                              osems_ref.at[2 * slot + 1]).start()

    for q in range(2, 4):
        rows = pl.ds(q * _QL, _QL)
        slot = q % 2
        pltpu.make_async_copy(hbuf_ref.at[slot], h_ref.at[rows, :],
                              osems_ref.at[2 * slot]).wait()
        pltpu.make_async_copy(cbuf_ref.at[slot], c_ref.at[rows, :],
                              osems_ref.at[2 * slot + 1]).wait()


@functools.partial(jax.jit, static_argnames=())
def kernel(adj_t, x_c, hidden, l_batch, W_ih, W_hh, b_ih, b_hh):
    del l_batch
    vmem = lambda: pl.BlockSpec(memory_space=pltpu.MemorySpace.VMEM)
    hbm = lambda: pl.BlockSpec(memory_space=pltpu.MemorySpace.HBM)
    h, c = pl.pallas_call(
        _fused_body,
        in_specs=[hbm(), hbm(), hbm(), vmem(), vmem(), vmem(), vmem()],
        out_specs=[hbm(), hbm()],
        out_shape=[jax.ShapeDtypeStruct((_N_L, _D), jnp.float32)] * 2,
        scratch_shapes=[
            pltpu.VMEM((_N_BUF, _CHUNK, _N_L), jnp.float32),
            pltpu.VMEM((_N_BUF, _CHUNK, _D), jnp.float32),
            pltpu.VMEM((2, _N_L, _D), jnp.float32),
            pltpu.VMEM((_N_L, _D), jnp.float32),
            pltpu.VMEM((_N_L, 4 * _D), jnp.float32),
            pltpu.VMEM((2, _QL, _D), jnp.float32),
            pltpu.VMEM((2, _QL, _D), jnp.float32),
            pltpu.SemaphoreType.DMA((_N_BUF,)),
            pltpu.SemaphoreType.DMA((_N_BUF,)),
            pltpu.SemaphoreType.DMA,
            pltpu.SemaphoreType.DMA,
            pltpu.SemaphoreType.DMA((4,)),
        ],
    )(adj_t, x_c, hidden, W_ih, W_hh,
      b_ih.reshape(1, 4 * _D), b_hh.reshape(1, 4 * _D))
    return (h, c)


# speedup vs baseline: 1.0342x; 1.0342x over previous
import functools

import jax
import jax.numpy as jnp
from jax.experimental import pallas as pl
from jax.experimental.pallas import tpu as pltpu

_N_C, _N_L, _D = 10000, 4096, 128
_CHUNK = 1000
_N_CHUNKS = _N_C // _CHUNK
_N_BUF = 2
_QL = _N_L // 4


def _fused_body(adj_ref, xc_ref, hid_ref, wih_ref, whh_ref, bih_ref,
                bhh_ref, h_ref, c_ref, bufs_ref, xcb_ref, hid_buf,
                acc_ref, gpart_ref, hbuf_ref, cbuf_ref, sems_ref,
                xsems_ref, hsem_ref, csem_ref, osems_ref):
    def start(i):
        slot = i % _N_BUF if isinstance(i, int) else jax.lax.rem(i, _N_BUF)
        pltpu.make_async_copy(
            adj_ref.at[pl.ds(i * _CHUNK, _CHUNK), :],
            bufs_ref.at[slot], sems_ref.at[slot]).start()
        pltpu.make_async_copy(
            xc_ref.at[pl.ds(i * _CHUNK, _CHUNK), :],
            xcb_ref.at[slot], xsems_ref.at[slot]).start()

    xl_cp = pltpu.make_async_copy(hid_ref.at[0], hid_buf.at[0], hsem_ref)
    xl_cp.start()
    for i in range(_N_BUF):
        start(i)
    c0_cp = pltpu.make_async_copy(hid_ref.at[1], hid_buf.at[1], csem_ref)
    c0_cp.start()
    xl_cp.wait()

    def mm_t(a, w):
        return jax.lax.dot_general(
            a, w, dimension_numbers=(((1,), (1,)), ((), ())),
            preferred_element_type=jnp.float32)

    bias = bih_ref[...] + bhh_ref[...]
    w_flip = wih_ref[:, _D:]

    for q in range(4):
        rows = pl.ds(q * _QL, _QL)
        frows = pl.ds(((q + 2) % 4) * _QL, _QL)
        gpart_ref[rows, :] = mm_t(hid_buf[0, frows, :], w_flip) \
            + mm_t(hid_buf[0, rows, :], whh_ref[...]) + bias
    pltpu.make_async_copy(
        adj_ref.at[pl.ds(0, _CHUNK), :], bufs_ref.at[0],
        sems_ref.at[0]).wait()
    pltpu.make_async_copy(
        xc_ref.at[pl.ds(0, _CHUNK), :], xcb_ref.at[0],
        xsems_ref.at[0]).wait()
    acc_ref[...] = jax.lax.dot_general(
        bufs_ref[0], xcb_ref[0],
        dimension_numbers=(((0,), (0,)), ((), ())),
        preferred_element_type=jnp.float32)
    start(_N_BUF)

    def step(i, _):
        slot = jax.lax.rem(i, _N_BUF)
        pltpu.make_async_copy(
            adj_ref.at[pl.ds(i * _CHUNK, _CHUNK), :],
            bufs_ref.at[slot], sems_ref.at[slot]).wait()
        pltpu.make_async_copy(
            xc_ref.at[pl.ds(i * _CHUNK, _CHUNK), :],
            xcb_ref.at[slot], xsems_ref.at[slot]).wait()
        acc_ref[...] += jax.lax.dot_general(
            bufs_ref[slot], xcb_ref[slot],
            dimension_numbers=(((0,), (0,)), ((), ())),
            preferred_element_type=jnp.float32)

        @pl.when(i + _N_BUF < _N_CHUNKS)
        def _refill():
            start(i + _N_BUF)
        return _

    jax.lax.fori_loop(1, _N_CHUNKS - 1, step, 0)

    last_slot = (_N_CHUNKS - 1) % _N_BUF
    pltpu.make_async_copy(
        adj_ref.at[pl.ds((_N_CHUNKS - 1) * _CHUNK, _CHUNK), :],
        bufs_ref.at[last_slot], sems_ref.at[last_slot]).wait()
    pltpu.make_async_copy(
        xc_ref.at[pl.ds((_N_CHUNKS - 1) * _CHUNK, _CHUNK), :],
        xcb_ref.at[last_slot], xsems_ref.at[last_slot]).wait()
    c0_cp.wait()

    w_msg = wih_ref[:, :_D]
    for q in range(4):
        rows = pl.ds(q * _QL, _QL)
        msg_q = acc_ref[rows, :] + jax.lax.dot_general(
            bufs_ref[last_slot, :, rows], xcb_ref[last_slot],
            dimension_numbers=(((0,), (0,)), ((), ())),
            preferred_element_type=jnp.float32)
        gates = gpart_ref[rows, :] + mm_t(msg_q, w_msg)
        i_g = jax.nn.sigmoid(gates[:, :_D])
        f_g = jax.nn.sigmoid(gates[:, _D:2 * _D])
        g_g = jnp.tanh(gates[:, 2 * _D:3 * _D])
        o_g = jax.nn.sigmoid(gates[:, 3 * _D:])
        c = f_g * hid_buf[1, rows, :] + i_g * g_g
        slot = q % 2
        if q >= 2:
            prows = pl.ds((q - 2) * _QL, _QL)
            pltpu.make_async_copy(hbuf_ref.at[slot], h_ref.at[prows, :],
                                  osems_ref.at[2 * slot]).wait()
            pltpu.make_async_copy(cbuf_ref.at[slot], c_ref.at[prows, :],
                                  osems_ref.at[2 * slot + 1]).wait()
        hbuf_ref[slot] = o_g * jnp.tanh(c)
        cbuf_ref[slot] = c
        pltpu.make_async_copy(hbuf_ref.at[slot], h_ref.at[rows, :],
                              osems_ref.at[2 * slot]).start()
        pltpu.make_async_copy(cbuf_ref.at[slot], c_ref.at[rows, :],
                              osems_ref.at[2 * slot + 1]).start()

    for q in range(2, 4):
        rows = pl.ds(q * _QL, _QL)
        slot = q % 2
        pltpu.make_async_copy(hbuf_ref.at[slot], h_ref.at[rows, :],
                              osems_ref.at[2 * slot]).wait()
        pltpu.make_async_copy(cbuf_ref.at[slot], c_ref.at[rows, :],
                              osems_ref.at[2 * slot + 1]).wait()


@functools.partial(jax.jit, static_argnames=())
def kernel(adj_t, x_c, hidden, l_batch, W_ih, W_hh, b_ih, b_hh):
    del l_batch
    vmem = lambda: pl.BlockSpec(memory_space=pltpu.MemorySpace.VMEM)
    hbm = lambda: pl.BlockSpec(memory_space=pltpu.MemorySpace.HBM)
    h, c = pl.pallas_call(
        _fused_body,
        in_specs=[hbm(), hbm(), hbm(), vmem(), vmem(), vmem(), vmem()],
        out_specs=[hbm(), hbm()],
        out_shape=[jax.ShapeDtypeStruct((_N_L, _D), jnp.float32)] * 2,
        scratch_shapes=[
            pltpu.VMEM((_N_BUF, _CHUNK, _N_L), jnp.float32),
            pltpu.VMEM((_N_BUF, _CHUNK, _D), jnp.float32),
            pltpu.VMEM((2, _N_L, _D), jnp.float32),
            pltpu.VMEM((_N_L, _D), jnp.float32),
            pltpu.VMEM((_N_L, 4 * _D), jnp.float32),
            pltpu.VMEM((2, _QL, _D), jnp.float32),
            pltpu.VMEM((2, _QL, _D), jnp.float32),
            pltpu.SemaphoreType.DMA((_N_BUF,)),
            pltpu.SemaphoreType.DMA((_N_BUF,)),
            pltpu.SemaphoreType.DMA,
            pltpu.SemaphoreType.DMA,
            pltpu.SemaphoreType.DMA((4,)),
        ],
    )(adj_t, x_c, hidden, W_ih, W_hh,
      b_ih.reshape(1, 4 * _D), b_hh.reshape(1, 4 * _D))
    return (h, c)
